# Initial kernel scaffold; baseline (speedup 1.0000x reference)
import jax
import jax.numpy as jnp
from jax.experimental import pallas as pl


def kernel(parents, pad_mask, bias):
    raise NotImplementedError("write your pallas kernel here")


# speedup vs baseline: 50.1348x; 50.1348x over previous
import jax
import jax.numpy as jnp
from jax.experimental import pallas as pl
from jax.experimental.pallas import tpu as pltpu

_H = 16
_NB = 8
_MAXA = 6
_NEG = float(jnp.finfo(jnp.float32).min)
_TILE = 64


def _bias_kernel(parents_ref, parentsT_ref, padk_ref, padq_ref, bias_ref,
                 out_ref, frow_scr, fcol_scr):
    L = parents_ref.shape[-1]
    t = pl.program_id(1)

    @pl.when(t == 0)
    def _compute_chains():
        p_row = parents_ref[0]
        p_col = parentsT_ref[0]
        p_row_f = p_row.astype(jnp.float32)
        p_col_f = p_col.astype(jnp.float32)
        iota_row = jax.lax.broadcasted_iota(jnp.int32, (1, L), 1)
        iota_col = jax.lax.broadcasted_iota(jnp.int32, (L, 1), 0)
        frow_scr[1:2, :] = p_row
        fcol_scr[:, 1:2] = p_col
        cur_row, cur_col = p_row, p_col
        for a in range(2, _MAXA + 1):
            onehot_t = (iota_col == cur_row).astype(jnp.float32)
            cur_row = jnp.dot(p_row_f, onehot_t,
                              preferred_element_type=jnp.float32,
                              precision=jax.lax.Precision.HIGHEST
                              ).astype(jnp.int32)
            onehot = (cur_col == iota_row).astype(jnp.float32)
            cur_col = jnp.dot(onehot, p_col_f,
                              preferred_element_type=jnp.float32,
                              precision=jax.lax.Precision.HIGHEST
                              ).astype(jnp.int32)
            frow_scr[a:a + 1, :] = cur_row
            fcol_scr[:, a:a + 1] = cur_col

    base = t * _TILE
    iota_row = jax.lax.broadcasted_iota(jnp.int32, (1, L), 1)
    iota_tile = base + jax.lax.broadcasted_iota(jnp.int32, (_TILE, 1), 0)

    fcols = [iota_tile] + [fcol_scr[pl.ds(base, _TILE), a:a + 1]
                           for a in range(1, _MAXA + 1)]
    frows = [iota_row] + [frow_scr[a:a + 1, :] for a in range(1, _MAXA + 1)]

    bucket = jnp.full((_TILE, L), _NB - 1, jnp.int32)
    for s in range(_MAXA, -1, -1):
        for a in range(0, s + 1):
            m = fcols[a] == frows[s - a]
            bucket = jnp.where(m, s, bucket)

    m0 = (bucket & 1) != 0
    m1 = (bucket & 2) != 0
    m2 = (bucket & 4) != 0
    vq = padq_ref[0, pl.ds(base, _TILE), :] > 0
    vk = padk_ref[0] > 0
    valid = jnp.logical_and(vq, vk)

    for h in range(_H):
        bv = [bias_ref[h, k] for k in range(_NB)]
        v01 = jnp.where(m0, bv[1], bv[0])
        v23 = jnp.where(m0, bv[3], bv[2])
        v45 = jnp.where(m0, bv[5], bv[4])
        v67 = jnp.where(m0, bv[7], bv[6])
        v0123 = jnp.where(m1, v23, v01)
        v4567 = jnp.where(m1, v67, v45)
        v = jnp.where(m2, v4567, v0123)
        out_ref[0, h] = jnp.where(valid, v, _NEG)


def kernel(parents, pad_mask, bias):
    B, L = parents.shape
    T = L // _TILE
    p_row = parents.reshape(B, 1, L)
    p_col = parents.reshape(B, L, 1)
    mk = pad_mask.astype(jnp.float32).reshape(B, 1, L)
    mq = pad_mask.astype(jnp.float32).reshape(B, L, 1)

    return pl.pallas_call(
        _bias_kernel,
        grid=(B, T),
        in_specs=[
            pl.BlockSpec((1, 1, L), lambda b, t: (b, 0, 0)),
            pl.BlockSpec((1, L, 1), lambda b, t: (b, 0, 0)),
            pl.BlockSpec((1, 1, L), lambda b, t: (b, 0, 0)),
            pl.BlockSpec((1, L, 1), lambda b, t: (b, 0, 0)),
            pl.BlockSpec(memory_space=pltpu.SMEM),
        ],
        out_specs=pl.BlockSpec((1, _H, _TILE, L), lambda b, t: (b, 0, t, 0)),
        out_shape=jax.ShapeDtypeStruct((B, _H, L, L), jnp.float32),
        scratch_shapes=[pltpu.VMEM((_NB, L), jnp.int32),
                        pltpu.VMEM((L, _NB), jnp.int32)],
        compiler_params=pltpu.CompilerParams(
            dimension_semantics=("arbitrary", "arbitrary"),
        ),
    )(p_row, p_col, mk, mq, bias)
